# baseline (speedup 1.0000x reference)
import jax
import jax.numpy as jnp
from jax import lax
from jax.experimental import pallas as pl
from jax.experimental.pallas import tpu as pltpu

E = 8
KTOP = 2
NGROUP = 4
GSZ = E // NGROUP
SCALE = 2.5
H = 1024
I = 512
SI = 1024
T = 2048

_TCHUNK = 512


def _silu(v):
    return v / (1.0 + jnp.exp(-v))


def _top1_mask(vals, width):
    m = jnp.max(vals, axis=1, keepdims=True)
    io = lax.broadcasted_iota(jnp.int32, vals.shape, 1)
    idx = jnp.min(jnp.where(vals == m, io, width), axis=1, keepdims=True)
    return io == idx


def _router_body(logits_ref, bias_ref, comb_ref):
    logits = logits_ref[...]
    scores = 1.0 / (1.0 + jnp.exp(-logits))
    swb = scores + bias_ref[...]

    e_i = lax.broadcasted_iota(jnp.int32, (E, NGROUP), 0)
    g_i = lax.broadcasted_iota(jnp.int32, (E, NGROUP), 1)
    m_eg = ((e_i // GSZ) == g_i).astype(jnp.float32)
    g_j = lax.broadcasted_iota(jnp.int32, (NGROUP, E), 0)
    e_j = lax.broadcasted_iota(jnp.int32, (NGROUP, E), 1)
    m_ge = ((e_j // GSZ) == g_j).astype(jnp.float32)

    gs = jnp.dot(swb, m_eg, preferred_element_type=jnp.float32, precision=lax.Precision.HIGHEST)
    p1 = _top1_mask(gs, NGROUP)
    p2 = _top1_mask(jnp.where(p1, -1e30, gs), NGROUP)
    gmask = jnp.logical_or(p1, p2).astype(jnp.float32)
    emask = jnp.dot(gmask, m_ge, preferred_element_type=jnp.float32, precision=lax.Precision.HIGHEST)

    masked = jnp.where(emask > 0.5, swb, -1e9)
    oh1 = _top1_mask(masked, E)
    oh2 = _top1_mask(jnp.where(oh1, -1e30, masked), E)
    w1 = jnp.sum(jnp.where(oh1, scores, 0.0), axis=1, keepdims=True)
    w2 = jnp.sum(jnp.where(oh2, scores, 0.0), axis=1, keepdims=True)
    r = SCALE / (w1 + w2 + 1e-20)
    comb_ref[...] = (jnp.where(oh1, w1, 0.0) + jnp.where(oh2, w2, 0.0)) * r


def _moe_body(comb_ref, x_ref, wg_ref, wu_ref, wd_ref, out_ref):
    e = pl.program_id(0)

    @pl.when(e == 0)
    def _init():
        out_ref[...] = jnp.zeros_like(out_ref)

    wg = wg_ref[0]
    wu = wu_ref[0]
    wd = wd_ref[0]
    lane = lax.broadcasted_iota(jnp.int32, (_TCHUNK, E), 1)
    for c in range(T // _TCHUNK):
        sl = pl.ds(c * _TCHUNK, _TCHUNK)
        x = x_ref[sl, :]
        g = jnp.dot(x, wg, preferred_element_type=jnp.float32)
        u = jnp.dot(x, wu, preferred_element_type=jnp.float32)
        a = (_silu(g) * u).astype(jnp.bfloat16)
        d = jnp.dot(a, wd, preferred_element_type=jnp.float32)
        cb = comb_ref[sl, :]
        col = jnp.sum(jnp.where(lane == e, cb, 0.0), axis=1, keepdims=True)
        out_ref[sl, :] += col * d


def _shared_body(x_ref, wsg_ref, wsu_ref, wsd_ref, routed_ref, out_ref):
    x = x_ref[...]
    g = jnp.dot(x, wsg_ref[...], preferred_element_type=jnp.float32)
    u = jnp.dot(x, wsu_ref[...], preferred_element_type=jnp.float32)
    a = (_silu(g) * u).astype(jnp.bfloat16)
    d = jnp.dot(a, wsd_ref[...], preferred_element_type=jnp.float32)
    out_ref[...] = routed_ref[...] + d


def kernel(hidden_states, gate_weight, e_score_correction_bias,
           w_gate, w_up, w_down, ws_gate, ws_up, ws_down):
    x = hidden_states
    logits = jnp.dot(x, gate_weight.T).astype(jnp.float32)
    bias2 = e_score_correction_bias.reshape(1, E)

    comb = pl.pallas_call(
        _router_body,
        out_shape=jax.ShapeDtypeStruct((T, E), jnp.float32),
        in_specs=[
            pl.BlockSpec((T, E), lambda: (0, 0)),
            pl.BlockSpec((1, E), lambda: (0, 0)),
        ],
        out_specs=pl.BlockSpec((T, E), lambda: (0, 0)),
    )(logits, bias2)

    xb = x.astype(jnp.bfloat16)
    wgb = w_gate.astype(jnp.bfloat16)
    wub = w_up.astype(jnp.bfloat16)
    wdb = w_down.astype(jnp.bfloat16)
    wsgb = ws_gate.astype(jnp.bfloat16)
    wsub = ws_up.astype(jnp.bfloat16)
    wsdb = ws_down.astype(jnp.bfloat16)

    routed = pl.pallas_call(
        _moe_body,
        grid=(E,),
        out_shape=jax.ShapeDtypeStruct((T, H), jnp.float32),
        in_specs=[
            pl.BlockSpec((T, E), lambda e: (0, 0)),
            pl.BlockSpec((T, H), lambda e: (0, 0)),
            pl.BlockSpec((1, H, I), lambda e: (e, 0, 0)),
            pl.BlockSpec((1, H, I), lambda e: (e, 0, 0)),
            pl.BlockSpec((1, I, H), lambda e: (e, 0, 0)),
        ],
        out_specs=pl.BlockSpec((T, H), lambda e: (0, 0)),
        compiler_params=pltpu.CompilerParams(
            dimension_semantics=("arbitrary",),
        ),
    )(comb, xb, wgb, wub, wdb)

    tb = 512
    out = pl.pallas_call(
        _shared_body,
        grid=(T // tb,),
        out_shape=jax.ShapeDtypeStruct((T, H), jnp.float32),
        in_specs=[
            pl.BlockSpec((tb, H), lambda i: (i, 0)),
            pl.BlockSpec((H, SI), lambda i: (0, 0)),
            pl.BlockSpec((H, SI), lambda i: (0, 0)),
            pl.BlockSpec((SI, H), lambda i: (0, 0)),
            pl.BlockSpec((tb, H), lambda i: (i, 0)),
        ],
        out_specs=pl.BlockSpec((tb, H), lambda i: (i, 0)),
        compiler_params=pltpu.CompilerParams(
            dimension_semantics=("arbitrary",),
        ),
    )(xb, wsgb, wsub, wsdb, routed)

    return out


# speedup vs baseline: 1.2788x; 1.2788x over previous
import jax
import jax.numpy as jnp
from jax import lax
from jax.experimental import pallas as pl
from jax.experimental.pallas import tpu as pltpu

E = 8
KTOP = 2
NGROUP = 4
GSZ = E // NGROUP
SCALE = 2.5
H = 1024
I = 512
SI = 1024
T = 2048

_TCHUNK = 512


def _silu(v):
    return v / (1.0 + jnp.exp(-v))


def _top1_mask(vals, width):
    m = jnp.max(vals, axis=1, keepdims=True)
    io = lax.broadcasted_iota(jnp.int32, vals.shape, 1)
    idx = jnp.min(jnp.where(vals == m, io, width), axis=1, keepdims=True)
    return io == idx


def _router_body(logits_ref, bias_ref, comb_ref):
    logits = logits_ref[...]
    scores = 1.0 / (1.0 + jnp.exp(-logits))
    swb = scores + bias_ref[...]

    e_i = lax.broadcasted_iota(jnp.int32, (E, NGROUP), 0)
    g_i = lax.broadcasted_iota(jnp.int32, (E, NGROUP), 1)
    m_eg = ((e_i // GSZ) == g_i).astype(jnp.float32)
    g_j = lax.broadcasted_iota(jnp.int32, (NGROUP, E), 0)
    e_j = lax.broadcasted_iota(jnp.int32, (NGROUP, E), 1)
    m_ge = ((e_j // GSZ) == g_j).astype(jnp.float32)

    gs = jnp.dot(swb, m_eg, preferred_element_type=jnp.float32, precision=lax.Precision.HIGHEST)
    p1 = _top1_mask(gs, NGROUP)
    p2 = _top1_mask(jnp.where(p1, -1e30, gs), NGROUP)
    gmask = jnp.logical_or(p1, p2).astype(jnp.float32)
    emask = jnp.dot(gmask, m_ge, preferred_element_type=jnp.float32, precision=lax.Precision.HIGHEST)

    masked = jnp.where(emask > 0.5, swb, -1e9)
    oh1 = _top1_mask(masked, E)
    oh2 = _top1_mask(jnp.where(oh1, -1e30, masked), E)
    w1 = jnp.sum(jnp.where(oh1, scores, 0.0), axis=1, keepdims=True)
    w2 = jnp.sum(jnp.where(oh2, scores, 0.0), axis=1, keepdims=True)
    r = SCALE / (w1 + w2 + 1e-20)
    comb_ref[...] = (jnp.where(oh1, w1, 0.0) + jnp.where(oh2, w2, 0.0)) * r


def _moe_body(comb_ref, x_ref, wg_ref, wu_ref, wd_ref, out_ref):
    e = pl.program_id(0)

    @pl.when(e == 0)
    def _init():
        out_ref[...] = jnp.zeros_like(out_ref)

    wg = wg_ref[0]
    wu = wu_ref[0]
    wd = wd_ref[0]
    lane = lax.broadcasted_iota(jnp.int32, (_TCHUNK, E), 1)
    for c in range(T // _TCHUNK):
        sl = pl.ds(c * _TCHUNK, _TCHUNK)
        x = x_ref[sl, :]
        g = jnp.dot(x, wg, preferred_element_type=jnp.float32)
        u = jnp.dot(x, wu, preferred_element_type=jnp.float32)
        a = (_silu(g) * u).astype(jnp.bfloat16)
        d = jnp.dot(a, wd, preferred_element_type=jnp.float32)
        cb = comb_ref[sl, :]
        col = jnp.sum(jnp.where(lane == e, cb, 0.0), axis=1, keepdims=True)
        out_ref[sl, :] += col * d


def _shared_body(x_ref, wsg_ref, wsu_ref, wsd_ref, routed_ref, out_ref):
    x = x_ref[...]
    g = jnp.dot(x, wsg_ref[...], preferred_element_type=jnp.float32)
    u = jnp.dot(x, wsu_ref[...], preferred_element_type=jnp.float32)
    a = (_silu(g) * u).astype(jnp.bfloat16)
    d = jnp.dot(a, wsd_ref[...], preferred_element_type=jnp.float32)
    out_ref[...] = routed_ref[...] + d


def kernel(hidden_states, gate_weight, e_score_correction_bias,
           w_gate, w_up, w_down, ws_gate, ws_up, ws_down):
    x = hidden_states
    logits = jnp.dot(x, gate_weight.T).astype(jnp.float32)
    bias2 = e_score_correction_bias.reshape(1, E)

    comb = pl.pallas_call(
        _router_body,
        out_shape=jax.ShapeDtypeStruct((T, E), jnp.float32),
        in_specs=[
            pl.BlockSpec((T, E), lambda: (0, 0)),
            pl.BlockSpec((1, E), lambda: (0, 0)),
        ],
        out_specs=pl.BlockSpec((T, E), lambda: (0, 0)),
    )(logits, bias2)

    routed = pl.pallas_call(
        _moe_body,
        grid=(E,),
        out_shape=jax.ShapeDtypeStruct((T, H), jnp.float32),
        in_specs=[
            pl.BlockSpec((T, E), lambda e: (0, 0)),
            pl.BlockSpec((T, H), lambda e: (0, 0)),
            pl.BlockSpec((1, H, I), lambda e: (e, 0, 0)),
            pl.BlockSpec((1, H, I), lambda e: (e, 0, 0)),
            pl.BlockSpec((1, I, H), lambda e: (e, 0, 0)),
        ],
        out_specs=pl.BlockSpec((T, H), lambda e: (0, 0)),
        compiler_params=pltpu.CompilerParams(
            dimension_semantics=("arbitrary",),
        ),
    )(comb, x, w_gate, w_up, w_down)

    tb = 512
    out = pl.pallas_call(
        _shared_body,
        grid=(T // tb,),
        out_shape=jax.ShapeDtypeStruct((T, H), jnp.float32),
        in_specs=[
            pl.BlockSpec((tb, H), lambda i: (i, 0)),
            pl.BlockSpec((H, SI), lambda i: (0, 0)),
            pl.BlockSpec((H, SI), lambda i: (0, 0)),
            pl.BlockSpec((SI, H), lambda i: (0, 0)),
            pl.BlockSpec((tb, H), lambda i: (i, 0)),
        ],
        out_specs=pl.BlockSpec((tb, H), lambda i: (i, 0)),
        compiler_params=pltpu.CompilerParams(
            dimension_semantics=("arbitrary",),
        ),
    )(x, ws_gate, ws_up, ws_down, routed)

    return out


# speedup vs baseline: 1.3891x; 1.0863x over previous
import jax
import jax.numpy as jnp
from jax import lax
from jax.experimental import pallas as pl
from jax.experimental.pallas import tpu as pltpu

E = 8
NGROUP = 4
GSZ = E // NGROUP
SCALE = 2.5
H = 1024
I = 512
SI = 1024
T = 2048

_TCHUNK = 512


def _silu(v):
    return v / (1.0 + jnp.exp(-v))


def _top1_mask(vals, width):
    m = jnp.max(vals, axis=1, keepdims=True)
    io = lax.broadcasted_iota(jnp.int32, vals.shape, 1)
    idx = jnp.min(jnp.where(vals == m, io, width), axis=1, keepdims=True)
    return io == idx


def _routing(logits, bias):
    scores = 1.0 / (1.0 + jnp.exp(-logits))
    swb = scores + bias

    e_i = lax.broadcasted_iota(jnp.int32, (E, NGROUP), 0)
    g_i = lax.broadcasted_iota(jnp.int32, (E, NGROUP), 1)
    m_eg = ((e_i // GSZ) == g_i).astype(jnp.float32)
    g_j = lax.broadcasted_iota(jnp.int32, (NGROUP, E), 0)
    e_j = lax.broadcasted_iota(jnp.int32, (NGROUP, E), 1)
    m_ge = ((e_j // GSZ) == g_j).astype(jnp.float32)

    gs = jnp.dot(swb, m_eg, preferred_element_type=jnp.float32,
                 precision=lax.Precision.HIGHEST)
    p1 = _top1_mask(gs, NGROUP)
    p2 = _top1_mask(jnp.where(p1, -1e30, gs), NGROUP)
    gmask = jnp.logical_or(p1, p2).astype(jnp.float32)
    emask = jnp.dot(gmask, m_ge, preferred_element_type=jnp.float32,
                    precision=lax.Precision.HIGHEST)

    masked = jnp.where(emask > 0.5, swb, -1e9)
    oh1 = _top1_mask(masked, E)
    oh2 = _top1_mask(jnp.where(oh1, -1e30, masked), E)
    w1 = jnp.sum(jnp.where(oh1, scores, 0.0), axis=1, keepdims=True)
    w2 = jnp.sum(jnp.where(oh2, scores, 0.0), axis=1, keepdims=True)
    r = SCALE / (w1 + w2 + 1e-20)
    return (jnp.where(oh1, w1, 0.0) + jnp.where(oh2, w2, 0.0)) * r


def _mlp_accum(x_ref, wg, wu, wd, comb_ref, out_ref, e, weighted):
    lane = lax.broadcasted_iota(jnp.int32, (_TCHUNK, E), 1)
    for c in range(T // _TCHUNK):
        sl = pl.ds(c * _TCHUNK, _TCHUNK)
        x = x_ref[sl, :]
        g = jnp.dot(x, wg, preferred_element_type=jnp.float32)
        u = jnp.dot(x, wu, preferred_element_type=jnp.float32)
        a = _silu(g) * u
        d = jnp.dot(a, wd, preferred_element_type=jnp.float32)
        if weighted:
            cb = comb_ref[sl, :]
            col = jnp.sum(jnp.where(lane == e, cb, 0.0), axis=1, keepdims=True)
            out_ref[sl, :] += col * d
        else:
            out_ref[sl, :] += d


def _fused_body(logits_ref, bias_ref, x_ref, wg_ref, wu_ref, wd_ref,
                wsg_ref, wsu_ref, wsd_ref, out_ref, comb_ref):
    e = pl.program_id(0)

    @pl.when(e == 0)
    def _init():
        out_ref[...] = jnp.zeros_like(out_ref)
        comb_ref[...] = _routing(logits_ref[...], bias_ref[...])

    @pl.when(e < E)
    def _experts():
        _mlp_accum(x_ref, wg_ref[0], wu_ref[0], wd_ref[0],
                   comb_ref, out_ref, e, weighted=True)

    @pl.when(e >= E)
    def _shared():
        _mlp_accum(x_ref, wsg_ref[...], wsu_ref[...], wsd_ref[...],
                   comb_ref, out_ref, e, weighted=False)


def kernel(hidden_states, gate_weight, e_score_correction_bias,
           w_gate, w_up, w_down, ws_gate, ws_up, ws_down):
    x = hidden_states
    logits = jnp.dot(x, gate_weight.T).astype(jnp.float32)
    bias2 = e_score_correction_bias.reshape(1, E)

    out = pl.pallas_call(
        _fused_body,
        grid=(E + 2,),
        out_shape=jax.ShapeDtypeStruct((T, H), jnp.float32),
        in_specs=[
            pl.BlockSpec((T, E), lambda e: (0, 0)),
            pl.BlockSpec((1, E), lambda e: (0, 0)),
            pl.BlockSpec((T, H), lambda e: (0, 0)),
            pl.BlockSpec((1, H, I), lambda e: (jnp.minimum(e, E - 1), 0, 0)),
            pl.BlockSpec((1, H, I), lambda e: (jnp.minimum(e, E - 1), 0, 0)),
            pl.BlockSpec((1, I, H), lambda e: (jnp.minimum(e, E - 1), 0, 0)),
            pl.BlockSpec((H, I), lambda e: (0, jnp.clip(e - E, 0, 1))),
            pl.BlockSpec((H, I), lambda e: (0, jnp.clip(e - E, 0, 1))),
            pl.BlockSpec((I, H), lambda e: (jnp.clip(e - E, 0, 1), 0)),
        ],
        out_specs=pl.BlockSpec((T, H), lambda e: (0, 0)),
        scratch_shapes=[pltpu.VMEM((T, E), jnp.float32)],
        compiler_params=pltpu.CompilerParams(
            dimension_semantics=("arbitrary",),
        ),
    )(logits, bias2, x, w_gate, w_up, w_down, ws_gate, ws_up, ws_down)

    return out
